# initial kernel scaffold (unmeasured)
import jax
import jax.numpy as jnp
from jax import lax
from jax.experimental import pallas as pl
from jax.experimental.pallas import tpu as pltpu


def kernel(
    t,
):
    def body(*refs):
        pass

    out_shape = jax.ShapeDtypeStruct(..., jnp.float32)
    return pl.pallas_call(body, out_shape=out_shape)(...)


# baseline (device time: 42495 ns/iter reference)
import jax
import jax.numpy as jnp
from jax import lax
from jax.experimental import pallas as pl
from jax.experimental.pallas import tpu as pltpu

N_DEV = 16
STEPS = 4

try:
    _ds = jax.devices()
    print(
        "[topo] n=", len(_ds),
        [(d.id, getattr(d, "coords", None), getattr(d, "core_on_chip", None))
         for d in _ds],
        flush=True,
    )
except Exception as _e:
    print("[topo] probe failed:", _e, flush=True)


def kernel(t):
    m, n = t.shape

    def body(x_ref, out_ref, accum_ref, recv_ref, send_sems, recv_sems):
        my = lax.axis_index("i")

        barrier_sem = pltpu.get_barrier_semaphore()
        for s in range(STEPS):
            partner = my ^ (1 << s)
            pl.semaphore_signal(
                barrier_sem,
                inc=1,
                device_id=(partner,),
                device_id_type=pl.DeviceIdType.MESH,
            )
        pl.semaphore_wait(barrier_sem, STEPS)

        accum_ref[...] = x_ref[...]

        for s in range(STEPS):
            partner = my ^ (1 << s)
            rdma = pltpu.make_async_remote_copy(
                src_ref=accum_ref,
                dst_ref=recv_ref.at[s],
                send_sem=send_sems.at[s],
                recv_sem=recv_sems.at[s],
                device_id=(partner,),
                device_id_type=pl.DeviceIdType.MESH,
            )
            rdma.start()
            rdma.wait()
            accum_ref[...] = accum_ref[...] + recv_ref[s]

        sv = accum_ref[...]
        r = jnp.maximum(sv, 0.0)
        out_ref[...] = jnp.tanh(sv) * sv * sv + r * r * r

    return pl.pallas_call(
        body,
        out_shape=jax.ShapeDtypeStruct((m, n), jnp.float32),
        in_specs=[pl.BlockSpec(memory_space=pltpu.VMEM)],
        out_specs=pl.BlockSpec(memory_space=pltpu.VMEM),
        scratch_shapes=[
            pltpu.VMEM((m, n), jnp.float32),
            pltpu.VMEM((STEPS, m, n), jnp.float32),
            pltpu.SemaphoreType.DMA((STEPS,)),
            pltpu.SemaphoreType.DMA((STEPS,)),
        ],
        compiler_params=pltpu.CompilerParams(collective_id=0),
    )(t)


if __name__ == "__main__":
    pass


# device time: 21082 ns/iter; 2.0157x vs baseline; 2.0157x over previous
import jax
import jax.numpy as jnp
from jax import lax
from jax.experimental import pallas as pl
from jax.experimental.pallas import tpu as pltpu

N_DEV = 16


def kernel(t):
    m, n = t.shape
    mc = m // N_DEV

    def body(x_ref, out_ref, recv_ref, chunk_ref,
             send1_sems, recv1_sems, send2_sems, recv2_sems):
        my = lax.axis_index("i")

        barrier_sem = pltpu.get_barrier_semaphore()
        for j in range(N_DEV):
            @pl.when(my != j)
            def _():
                pl.semaphore_signal(
                    barrier_sem, inc=1,
                    device_id=(j,), device_id_type=pl.DeviceIdType.MESH,
                )
        pl.semaphore_wait(barrier_sem, N_DEV - 1)

        for j in range(N_DEV):
            @pl.when(my != j)
            def _():
                rdma = pltpu.make_async_remote_copy(
                    src_ref=x_ref.at[pl.ds(j * mc, mc)],
                    dst_ref=recv_ref.at[my],
                    send_sem=send1_sems.at[j],
                    recv_sem=recv1_sems.at[my],
                    device_id=(j,),
                    device_id_type=pl.DeviceIdType.MESH,
                )
                rdma.start()

        recv_ref[my] = x_ref[pl.ds(my * mc, mc)]

        for k in range(N_DEV):
            @pl.when(my != k)
            def _():
                recv = pltpu.make_async_remote_copy(
                    src_ref=x_ref.at[pl.ds(0, mc)],
                    dst_ref=recv_ref.at[k],
                    send_sem=send1_sems.at[k],
                    recv_sem=recv1_sems.at[k],
                    device_id=(k,),
                    device_id_type=pl.DeviceIdType.MESH,
                )
                recv.wait_recv()

        sv = jnp.sum(recv_ref[...], axis=0)
        r = jnp.maximum(sv, 0.0)
        chunk_ref[...] = jnp.tanh(sv) * sv * sv + r * r * r
        out_ref[pl.ds(my * mc, mc)] = chunk_ref[...]

        for j in range(N_DEV):
            @pl.when(my != j)
            def _():
                rdma = pltpu.make_async_remote_copy(
                    src_ref=chunk_ref,
                    dst_ref=out_ref.at[pl.ds(my * mc, mc)],
                    send_sem=send2_sems.at[j],
                    recv_sem=recv2_sems.at[my],
                    device_id=(j,),
                    device_id_type=pl.DeviceIdType.MESH,
                )
                rdma.start()

        for k in range(N_DEV):
            @pl.when(my != k)
            def _():
                recv = pltpu.make_async_remote_copy(
                    src_ref=chunk_ref,
                    dst_ref=out_ref.at[pl.ds(k * mc, mc)],
                    send_sem=send2_sems.at[k],
                    recv_sem=recv2_sems.at[k],
                    device_id=(k,),
                    device_id_type=pl.DeviceIdType.MESH,
                )
                recv.wait_recv()

        for j in range(N_DEV):
            @pl.when(my != j)
            def _():
                send = pltpu.make_async_remote_copy(
                    src_ref=chunk_ref,
                    dst_ref=out_ref.at[pl.ds(0, mc)],
                    send_sem=send2_sems.at[j],
                    recv_sem=recv2_sems.at[my],
                    device_id=(j,),
                    device_id_type=pl.DeviceIdType.MESH,
                )
                send.wait_send()
                send1 = pltpu.make_async_remote_copy(
                    src_ref=x_ref.at[pl.ds(j * mc, mc)],
                    dst_ref=recv_ref.at[my],
                    send_sem=send1_sems.at[j],
                    recv_sem=recv1_sems.at[my],
                    device_id=(j,),
                    device_id_type=pl.DeviceIdType.MESH,
                )
                send1.wait_send()

    return pl.pallas_call(
        body,
        out_shape=jax.ShapeDtypeStruct((m, n), jnp.float32),
        in_specs=[pl.BlockSpec(memory_space=pltpu.VMEM)],
        out_specs=pl.BlockSpec(memory_space=pltpu.VMEM),
        scratch_shapes=[
            pltpu.VMEM((N_DEV, mc, n), jnp.float32),
            pltpu.VMEM((mc, n), jnp.float32),
            pltpu.SemaphoreType.DMA((N_DEV,)),
            pltpu.SemaphoreType.DMA((N_DEV,)),
            pltpu.SemaphoreType.DMA((N_DEV,)),
            pltpu.SemaphoreType.DMA((N_DEV,)),
        ],
        compiler_params=pltpu.CompilerParams(collective_id=0),
    )(t)
